# Initial kernel scaffold; baseline (speedup 1.0000x reference)
import jax
import jax.numpy as jnp
from jax.experimental import pallas as pl


def kernel(x_sequence, edge_index, batch_size, num_nodes, W_gnn0, b_gnn0, W_gnn1, b_gnn1, W_ih0, W_hh0, b_ih0, b_hh0, W_ih1, W_hh1, b_ih1, b_hh1, W_out, b_out):
    raise NotImplementedError("write your pallas kernel here")


# speedup vs baseline: 2.6841x; 2.6841x over previous
import functools

import jax
import jax.numpy as jnp
from jax import lax
from jax.experimental import pallas as pl
from jax.experimental.pallas import tpu as pltpu
from jax.experimental.pallas import tpu_sc as plsc

N = 10000
F = 128
E = 160000
NTILES = 32
EPT = E // NTILES
CHUNK = 128
NCHUNK = (EPT + CHUNK - 1) // CHUNK
EPT_PAD = NCHUNK * CHUNK
ROWS_PT = 624
REM_ROWS = N - 16 * ROWS_PT
REM0 = 16 * ROWS_PT
ACC_ROWS = N + 8


def _sc_edge_aggregate(table, zeros_tbl, src3, dst3):
  mesh = plsc.VectorSubcoreMesh(core_axis_name="c", subcore_axis_name="s")

  @functools.partial(
      pl.kernel,
      mesh=mesh,
      out_type=jax.ShapeDtypeStruct((2, N, F), jnp.float32),
      scratch_types=[
          pltpu.VMEM((NCHUNK, CHUNK), jnp.int32),
          pltpu.VMEM((NCHUNK, CHUNK), jnp.int32),
          pltpu.VMEM((CHUNK, F), jnp.float32),
          pltpu.VMEM_SHARED((ACC_ROWS, F), jnp.float32),
          pltpu.SemaphoreType.DMA,
      ],
  )
  def k(table_hbm, zeros_hbm, src_hbm, dst_hbm, out_hbm,
        sidx, didx, gbuf, acc, sem):
    c = lax.axis_index("c")
    s = lax.axis_index("s")
    wid = c * 16 + s
    row0 = s * ROWS_PT

    @pl.when(c == 0)
    def _():
      pltpu.sync_copy(table_hbm.at[pl.ds(row0, ROWS_PT)],
                      acc.at[pl.ds(row0, ROWS_PT)])

      @pl.when(s == 0)
      def _():
        pltpu.sync_copy(table_hbm.at[pl.ds(REM0, REM_ROWS)],
                        acc.at[pl.ds(REM0, REM_ROWS)])

    @pl.when(c != 0)
    def _():
      pltpu.sync_copy(zeros_hbm.at[pl.ds(row0, ROWS_PT)],
                      acc.at[pl.ds(row0, ROWS_PT)])

      @pl.when(s == 0)
      def _():
        pltpu.sync_copy(zeros_hbm.at[pl.ds(REM0, REM_ROWS)],
                        acc.at[pl.ds(REM0, REM_ROWS)])

    pltpu.sync_copy(src_hbm.at[wid], sidx)
    pltpu.sync_copy(dst_hbm.at[wid], didx)
    plsc.subcore_barrier()

    def body(i, carry):
      pltpu.async_copy(table_hbm.at[sidx.at[i]], gbuf, sem).wait()
      pltpu.sync_copy(gbuf, acc.at[didx.at[i]], add=True)
      return carry

    lax.fori_loop(0, NCHUNK, body, 0)
    plsc.subcore_barrier()

    pltpu.sync_copy(acc.at[pl.ds(row0, ROWS_PT)],
                    out_hbm.at[c, pl.ds(row0, ROWS_PT)])

    @pl.when(s == 0)
    def _():
      pltpu.sync_copy(acc.at[pl.ds(REM0, REM_ROWS)],
                      out_hbm.at[c, pl.ds(REM0, REM_ROWS)])

  return k(table, zeros_tbl, src3, dst3)


BN = 1000
NB = N // BN


def _dinv_kernel(d0_ref, d1_ref, o_ref):
  o_ref[...] = lax.rsqrt(d0_ref[0] + d1_ref[0])


def _compute_dinv(deg2):
  return pl.pallas_call(
      _dinv_kernel,
      grid=(NB,),
      in_specs=[
          pl.BlockSpec((1, BN, F), lambda j: (0, j, 0)),
          pl.BlockSpec((1, BN, F), lambda j: (1, j, 0)),
      ],
      out_specs=pl.BlockSpec((BN, F), lambda j: (j, 0)),
      out_shape=jax.ShapeDtypeStruct((N, F), jnp.float32),
  )(deg2, deg2)


def _xw_kernel(x_ref, w_ref, dinv_ref, o_ref):
  xw = jnp.dot(x_ref[0], w_ref[...], precision="highest",
               preferred_element_type=jnp.float32)
  o_ref[0] = xw * dinv_ref[...]


def _xw_prescale(x3, w, dinv):
  bt = x3.shape[0]
  return pl.pallas_call(
      _xw_kernel,
      grid=(bt, NB),
      in_specs=[
          pl.BlockSpec((1, BN, F), lambda i, j: (i, j, 0)),
          pl.BlockSpec((F, F), lambda i, j: (0, 0)),
          pl.BlockSpec((BN, F), lambda i, j: (j, 0)),
      ],
      out_specs=pl.BlockSpec((1, BN, F), lambda i, j: (i, j, 0)),
      out_shape=jax.ShapeDtypeStruct((bt, N, F), jnp.float32),
  )(x3, w, dinv)


def _mid_kernel(p0_ref, p1_ref, dinv_ref, b_ref, w_ref, o_ref):
  h = jnp.maximum(dinv_ref[...] * (p0_ref[0, 0] + p1_ref[0, 0])
                  + b_ref[...], 0.0)
  o_ref[0] = dinv_ref[...] * jnp.dot(h, w_ref[...], precision="highest",
                                     preferred_element_type=jnp.float32)


def _mid_layer(parts, dinv, b, w):
  bt = parts.shape[0]
  return pl.pallas_call(
      _mid_kernel,
      grid=(bt, NB),
      in_specs=[
          pl.BlockSpec((1, 1, BN, F), lambda i, j: (i, 0, j, 0)),
          pl.BlockSpec((1, 1, BN, F), lambda i, j: (i, 1, j, 0)),
          pl.BlockSpec((BN, F), lambda i, j: (j, 0)),
          pl.BlockSpec((1, F), lambda i, j: (0, 0)),
          pl.BlockSpec((F, F), lambda i, j: (0, 0)),
      ],
      out_specs=pl.BlockSpec((1, BN, F), lambda i, j: (i, j, 0)),
      out_shape=jax.ShapeDtypeStruct((bt, N, F), jnp.float32),
  )(parts, parts, dinv, b, w)


def _mean_kernel(p0_ref, p1_ref, dinv_ref, b_ref, o_ref):
  j = pl.program_id(1)

  @pl.when(j == 0)
  def _():
    o_ref[...] = jnp.zeros_like(o_ref)

  h = jnp.maximum(dinv_ref[...] * (p0_ref[0, 0] + p1_ref[0, 0])
                  + b_ref[...], 0.0)
  o_ref[0] += jnp.sum(h, axis=0, keepdims=True)

  @pl.when(j == NB - 1)
  def _():
    o_ref[...] *= (1.0 / N)


def _mean_layer(parts, dinv, b):
  bt = parts.shape[0]
  return pl.pallas_call(
      _mean_kernel,
      grid=(bt, NB),
      in_specs=[
          pl.BlockSpec((1, 1, BN, F), lambda i, j: (i, 0, j, 0)),
          pl.BlockSpec((1, 1, BN, F), lambda i, j: (i, 1, j, 0)),
          pl.BlockSpec((BN, F), lambda i, j: (j, 0)),
          pl.BlockSpec((1, F), lambda i, j: (0, 0)),
      ],
      out_specs=pl.BlockSpec((1, 1, F), lambda i, j: (i, 0, 0)),
      out_shape=jax.ShapeDtypeStruct((bt, 1, F), jnp.float32),
  )(parts, parts, dinv, b)


def _lstm_kernel(tin_ref, wi0_ref, wh0_ref, b0_ref, wi1_ref, wh1_ref, b1_ref,
                 wo_ref, bo_ref, o_ref):
  T = tin_ref.shape[1]
  Bp = tin_ref.shape[0]

  def run_layer(xs, wi_ref, wh_ref, b_ref):
    h = jnp.zeros((Bp, F), jnp.float32)
    c = jnp.zeros((Bp, F), jnp.float32)
    hs = []
    for t in range(T):
      gates = (jnp.dot(xs[t], wi_ref[...], precision="highest",
                       preferred_element_type=jnp.float32)
               + jnp.dot(h, wh_ref[...], precision="highest",
                         preferred_element_type=jnp.float32)
               + b_ref[...])
      i = jax.nn.sigmoid(gates[:, 0 * F:1 * F])
      f = jax.nn.sigmoid(gates[:, 1 * F:2 * F])
      g = jnp.tanh(gates[:, 2 * F:3 * F])
      o = jax.nn.sigmoid(gates[:, 3 * F:4 * F])
      c = f * c + i * g
      h = o * jnp.tanh(c)
      hs.append(h)
    return hs

  xs0 = [tin_ref[:, t, :] for t in range(T)]
  hs1 = run_layer(xs0, wi0_ref, wh0_ref, b0_ref)
  hs2 = run_layer(hs1, wi1_ref, wh1_ref, b1_ref)
  o_ref[...] = jnp.dot(hs2[-1], wo_ref[...], precision="highest",
                       preferred_element_type=jnp.float32) + bo_ref[...]


def _lstm_head(tin, wi0, wh0, b0, wi1, wh1, b1, wo, bo):
  Bp = tin.shape[0]
  return pl.pallas_call(
      _lstm_kernel,
      out_shape=jax.ShapeDtypeStruct((Bp, F), jnp.float32),
  )(tin, wi0, wh0, b0, wi1, wh1, b1, wo, bo)


def kernel(x_sequence, edge_index, batch_size, num_nodes,
           W_gnn0, b_gnn0, W_gnn1, b_gnn1,
           W_ih0, W_hh0, b_ih0, b_hh0, W_ih1, W_hh1, b_ih1, b_hh1,
           W_out, b_out):
  B, T, n, f = x_sequence.shape
  BT = B * T

  src = edge_index[0].reshape(NTILES, EPT)
  dst = edge_index[1].reshape(NTILES, EPT)
  pad = EPT_PAD - EPT
  src3 = jnp.concatenate(
      [src, jnp.zeros((NTILES, pad), jnp.int32)], axis=1
  ).reshape(NTILES, NCHUNK, CHUNK)
  dst3 = jnp.concatenate(
      [dst, jnp.full((NTILES, pad), N, jnp.int32)], axis=1
  ).reshape(NTILES, NCHUNK, CHUNK)

  zeros_tbl = jnp.zeros((N, F), jnp.float32)
  ones_tbl = jnp.ones((N, F), jnp.float32)

  deg2 = _sc_edge_aggregate(ones_tbl, zeros_tbl, src3, dst3)
  dinv = _compute_dinv(deg2)

  x3 = x_sequence.reshape(BT, n, f)
  x1p = _xw_prescale(x3, W_gnn0, dinv)

  def sc_step(carry, tbl):
    return carry, _sc_edge_aggregate(tbl, zeros_tbl, src3, dst3)

  _, parts1 = lax.scan(sc_step, 0, x1p)

  x2p = _mid_layer(parts1, dinv, b_gnn0.reshape(1, F), W_gnn1)
  _, parts2 = lax.scan(sc_step, 0, x2p)

  tmean = _mean_layer(parts2, dinv, b_gnn1.reshape(1, F))
  tin = tmean.reshape(B, T, F)

  tin_p = jnp.concatenate([tin, jnp.zeros_like(tin)], axis=0)
  bias0 = (b_ih0 + b_hh0).reshape(1, 4 * F)
  bias1 = (b_ih1 + b_hh1).reshape(1, 4 * F)
  row = _lstm_head(tin_p, W_ih0.T, W_hh0.T, bias0,
                   W_ih1.T, W_hh1.T, bias1, W_out, b_out.reshape(1, F))

  return jnp.broadcast_to(row[:B, None, :], (B, n, F))


# speedup vs baseline: 3.0514x; 1.1369x over previous
import functools

import jax
import jax.numpy as jnp
from jax import lax
from jax.experimental import pallas as pl
from jax.experimental.pallas import tpu as pltpu
from jax.experimental.pallas import tpu_sc as plsc

N = 10000
F = 128
E = 160000
NTILES = 32
EPT = E // NTILES
CHUNK = 128
NCHUNK = (EPT + CHUNK - 1) // CHUNK
EPT_PAD = NCHUNK * CHUNK
ROWS_PT = 624
REM_ROWS = N - 16 * ROWS_PT
REM0 = 16 * ROWS_PT
ACC_ROWS = N + 8


def _sc_edge_aggregate(table, zeros_tbl, src3, dst3):
  mesh = plsc.VectorSubcoreMesh(core_axis_name="c", subcore_axis_name="s")

  @functools.partial(
      pl.kernel,
      mesh=mesh,
      out_type=jax.ShapeDtypeStruct((2, N, F), jnp.float32),
      scratch_types=[
          pltpu.VMEM((NCHUNK, CHUNK), jnp.int32),
          pltpu.VMEM((NCHUNK, CHUNK), jnp.int32),
          pltpu.VMEM((CHUNK, F), jnp.float32),
          pltpu.VMEM((CHUNK, F), jnp.float32),
          pltpu.VMEM_SHARED((ACC_ROWS, F), jnp.float32),
          pltpu.SemaphoreType.DMA,
          pltpu.SemaphoreType.DMA,
      ],
  )
  def k(table_hbm, zeros_hbm, src_hbm, dst_hbm, out_hbm,
        sidx, didx, gbuf0, gbuf1, acc, sem0, sem1):
    c = lax.axis_index("c")
    s = lax.axis_index("s")
    wid = c * 16 + s
    row0 = s * ROWS_PT

    @pl.when(c == 0)
    def _():
      pltpu.sync_copy(table_hbm.at[pl.ds(row0, ROWS_PT)],
                      acc.at[pl.ds(row0, ROWS_PT)])

      @pl.when(s == 0)
      def _():
        pltpu.sync_copy(table_hbm.at[pl.ds(REM0, REM_ROWS)],
                        acc.at[pl.ds(REM0, REM_ROWS)])

    @pl.when(c != 0)
    def _():
      pltpu.sync_copy(zeros_hbm.at[pl.ds(row0, ROWS_PT)],
                      acc.at[pl.ds(row0, ROWS_PT)])

      @pl.when(s == 0)
      def _():
        pltpu.sync_copy(zeros_hbm.at[pl.ds(REM0, REM_ROWS)],
                        acc.at[pl.ds(REM0, REM_ROWS)])

    pltpu.sync_copy(src_hbm.at[wid], sidx)
    pltpu.sync_copy(dst_hbm.at[wid], didx)
    plsc.subcore_barrier()

    gbufs = (gbuf0, gbuf1)
    sems = (sem0, sem1)
    pltpu.make_async_copy(table_hbm.at[sidx.at[0]], gbuf0, sem0).start()
    pltpu.make_async_copy(table_hbm.at[sidx.at[1]], gbuf1, sem1).start()

    def body(j, carry):
      for b in range(2):
        i = 2 * j + b
        pltpu.make_async_copy(table_hbm.at[sidx.at[i]], gbufs[b],
                              sems[b]).wait()
        pltpu.sync_copy(gbufs[b], acc.at[didx.at[i]], add=True)
        pltpu.make_async_copy(table_hbm.at[sidx.at[i + 2]], gbufs[b],
                              sems[b]).start()
      return carry

    lax.fori_loop(0, NCHUNK // 2 - 1, body, 0)
    for b in range(2):
      i = NCHUNK - 2 + b
      pltpu.make_async_copy(table_hbm.at[sidx.at[i]], gbufs[b],
                            sems[b]).wait()
      pltpu.sync_copy(gbufs[b], acc.at[didx.at[i]], add=True)
    plsc.subcore_barrier()

    pltpu.sync_copy(acc.at[pl.ds(row0, ROWS_PT)],
                    out_hbm.at[c, pl.ds(row0, ROWS_PT)])

    @pl.when(s == 0)
    def _():
      pltpu.sync_copy(acc.at[pl.ds(REM0, REM_ROWS)],
                      out_hbm.at[c, pl.ds(REM0, REM_ROWS)])

  return k(table, zeros_tbl, src3, dst3)


BN = 1000
NB = N // BN


def _dinv_kernel(d0_ref, d1_ref, o_ref):
  o_ref[...] = lax.rsqrt(d0_ref[0] + d1_ref[0])


def _compute_dinv(deg2):
  return pl.pallas_call(
      _dinv_kernel,
      grid=(NB,),
      in_specs=[
          pl.BlockSpec((1, BN, F), lambda j: (0, j, 0)),
          pl.BlockSpec((1, BN, F), lambda j: (1, j, 0)),
      ],
      out_specs=pl.BlockSpec((BN, F), lambda j: (j, 0)),
      out_shape=jax.ShapeDtypeStruct((N, F), jnp.float32),
  )(deg2, deg2)


def _xw_kernel(x_ref, w_ref, dinv_ref, o_ref):
  xw = jnp.dot(x_ref[0], w_ref[...], precision="highest",
               preferred_element_type=jnp.float32)
  o_ref[0] = xw * dinv_ref[...]


def _xw_prescale(x3, w, dinv):
  bt = x3.shape[0]
  return pl.pallas_call(
      _xw_kernel,
      grid=(bt, NB),
      in_specs=[
          pl.BlockSpec((1, BN, F), lambda i, j: (i, j, 0)),
          pl.BlockSpec((F, F), lambda i, j: (0, 0)),
          pl.BlockSpec((BN, F), lambda i, j: (j, 0)),
      ],
      out_specs=pl.BlockSpec((1, BN, F), lambda i, j: (i, j, 0)),
      out_shape=jax.ShapeDtypeStruct((bt, N, F), jnp.float32),
  )(x3, w, dinv)


def _mid_kernel(p0_ref, p1_ref, dinv_ref, b_ref, w_ref, o_ref):
  h = jnp.maximum(dinv_ref[...] * (p0_ref[0, 0] + p1_ref[0, 0])
                  + b_ref[...], 0.0)
  o_ref[0] = dinv_ref[...] * jnp.dot(h, w_ref[...], precision="highest",
                                     preferred_element_type=jnp.float32)


def _mid_layer(parts, dinv, b, w):
  bt = parts.shape[0]
  return pl.pallas_call(
      _mid_kernel,
      grid=(bt, NB),
      in_specs=[
          pl.BlockSpec((1, 1, BN, F), lambda i, j: (i, 0, j, 0)),
          pl.BlockSpec((1, 1, BN, F), lambda i, j: (i, 1, j, 0)),
          pl.BlockSpec((BN, F), lambda i, j: (j, 0)),
          pl.BlockSpec((1, F), lambda i, j: (0, 0)),
          pl.BlockSpec((F, F), lambda i, j: (0, 0)),
      ],
      out_specs=pl.BlockSpec((1, BN, F), lambda i, j: (i, j, 0)),
      out_shape=jax.ShapeDtypeStruct((bt, N, F), jnp.float32),
  )(parts, parts, dinv, b, w)


def _mean_kernel(p0_ref, p1_ref, dinv_ref, b_ref, o_ref):
  j = pl.program_id(1)

  @pl.when(j == 0)
  def _():
    o_ref[...] = jnp.zeros_like(o_ref)

  h = jnp.maximum(dinv_ref[...] * (p0_ref[0, 0] + p1_ref[0, 0])
                  + b_ref[...], 0.0)
  o_ref[0] += jnp.sum(h, axis=0, keepdims=True)

  @pl.when(j == NB - 1)
  def _():
    o_ref[...] *= (1.0 / N)


def _mean_layer(parts, dinv, b):
  bt = parts.shape[0]
  return pl.pallas_call(
      _mean_kernel,
      grid=(bt, NB),
      in_specs=[
          pl.BlockSpec((1, 1, BN, F), lambda i, j: (i, 0, j, 0)),
          pl.BlockSpec((1, 1, BN, F), lambda i, j: (i, 1, j, 0)),
          pl.BlockSpec((BN, F), lambda i, j: (j, 0)),
          pl.BlockSpec((1, F), lambda i, j: (0, 0)),
      ],
      out_specs=pl.BlockSpec((1, 1, F), lambda i, j: (i, 0, 0)),
      out_shape=jax.ShapeDtypeStruct((bt, 1, F), jnp.float32),
  )(parts, parts, dinv, b)


def _lstm_kernel(tin_ref, wi0_ref, wh0_ref, b0_ref, wi1_ref, wh1_ref, b1_ref,
                 wo_ref, bo_ref, o_ref):
  T = tin_ref.shape[1]
  Bp = tin_ref.shape[0]

  def run_layer(xs, wi_ref, wh_ref, b_ref):
    h = jnp.zeros((Bp, F), jnp.float32)
    c = jnp.zeros((Bp, F), jnp.float32)
    hs = []
    for t in range(T):
      gates = (jnp.dot(xs[t], wi_ref[...], precision="highest",
                       preferred_element_type=jnp.float32)
               + jnp.dot(h, wh_ref[...], precision="highest",
                         preferred_element_type=jnp.float32)
               + b_ref[...])
      i = jax.nn.sigmoid(gates[:, 0 * F:1 * F])
      f = jax.nn.sigmoid(gates[:, 1 * F:2 * F])
      g = jnp.tanh(gates[:, 2 * F:3 * F])
      o = jax.nn.sigmoid(gates[:, 3 * F:4 * F])
      c = f * c + i * g
      h = o * jnp.tanh(c)
      hs.append(h)
    return hs

  xs0 = [tin_ref[:, t, :] for t in range(T)]
  hs1 = run_layer(xs0, wi0_ref, wh0_ref, b0_ref)
  hs2 = run_layer(hs1, wi1_ref, wh1_ref, b1_ref)
  o_ref[...] = jnp.dot(hs2[-1], wo_ref[...], precision="highest",
                       preferred_element_type=jnp.float32) + bo_ref[...]


def _lstm_head(tin, wi0, wh0, b0, wi1, wh1, b1, wo, bo):
  Bp = tin.shape[0]
  return pl.pallas_call(
      _lstm_kernel,
      out_shape=jax.ShapeDtypeStruct((Bp, F), jnp.float32),
  )(tin, wi0, wh0, b0, wi1, wh1, b1, wo, bo)


def kernel(x_sequence, edge_index, batch_size, num_nodes,
           W_gnn0, b_gnn0, W_gnn1, b_gnn1,
           W_ih0, W_hh0, b_ih0, b_hh0, W_ih1, W_hh1, b_ih1, b_hh1,
           W_out, b_out):
  B, T, n, f = x_sequence.shape
  BT = B * T

  src = edge_index[0].reshape(NTILES, EPT)
  dst = edge_index[1].reshape(NTILES, EPT)
  pad = EPT_PAD - EPT
  src3 = jnp.concatenate(
      [src, jnp.zeros((NTILES, pad), jnp.int32)], axis=1
  ).reshape(NTILES, NCHUNK, CHUNK)
  dst3 = jnp.concatenate(
      [dst, jnp.full((NTILES, pad), N, jnp.int32)], axis=1
  ).reshape(NTILES, NCHUNK, CHUNK)

  zeros_tbl = jnp.zeros((N, F), jnp.float32)
  ones_tbl = jnp.ones((N, F), jnp.float32)

  deg2 = _sc_edge_aggregate(ones_tbl, zeros_tbl, src3, dst3)
  dinv = _compute_dinv(deg2)

  x3 = x_sequence.reshape(BT, n, f)
  x1p = _xw_prescale(x3, W_gnn0, dinv)

  def sc_step(carry, tbl):
    return carry, _sc_edge_aggregate(tbl, zeros_tbl, src3, dst3)

  _, parts1 = lax.scan(sc_step, 0, x1p)

  x2p = _mid_layer(parts1, dinv, b_gnn0.reshape(1, F), W_gnn1)
  _, parts2 = lax.scan(sc_step, 0, x2p)

  tmean = _mean_layer(parts2, dinv, b_gnn1.reshape(1, F))
  tin = tmean.reshape(B, T, F)

  tin_p = jnp.concatenate([tin, jnp.zeros_like(tin)], axis=0)
  bias0 = (b_ih0 + b_hh0).reshape(1, 4 * F)
  bias1 = (b_ih1 + b_hh1).reshape(1, 4 * F)
  row = _lstm_head(tin_p, W_ih0.T, W_hh0.T, bias0,
                   W_ih1.T, W_hh1.T, bias1, W_out, b_out.reshape(1, F))

  return jnp.broadcast_to(row[:B, None, :], (B, n, F))
